# baseline (speedup 1.0000x reference)
import jax
import jax.numpy as jnp
from jax import lax
from jax.experimental import pallas as pl

NG = 512
K = 32


def _fps_jax(xyz, npoint, key):
    B, N, _ = xyz.shape
    farthest0 = jax.random.randint(key, (B,), 0, N)
    dist0 = jnp.full((B, N), 1e10, dtype=xyz.dtype)

    def body(carry, _):
        distance, farthest = carry
        centroid = jnp.take_along_axis(xyz, farthest[:, None, None], axis=1)
        d = jnp.sum((xyz - centroid) ** 2, axis=-1)
        distance = jnp.minimum(distance, d)
        new_farthest = jnp.argmax(distance, axis=-1)
        return (distance, new_farthest), farthest

    _, cent = lax.scan(body, (dist0, farthest0), None, length=npoint)
    return jnp.transpose(cent)


def _recenter_body(nb_ref, c_ref, out_ref):
    nb = nb_ref[0].reshape(NG, K, 3)
    out_ref[0] = (nb - c_ref[0][:, None, :]).reshape(NG, K * 3)


def kernel(xyz):
    B, N, C = xyz.shape
    centroids = _fps_jax(xyz, NG, jax.random.fold_in(jax.random.key(0), 1))
    center = jnp.take_along_axis(xyz, centroids[:, :, None], axis=1)
    sqrdists = -2.0 * jnp.matmul(center, jnp.transpose(xyz, (0, 2, 1)))
    sqrdists = sqrdists + jnp.sum(center ** 2, axis=-1)[:, :, None]
    sqrdists = sqrdists + jnp.sum(xyz ** 2, axis=-1)[:, None, :]
    _, idx = lax.top_k(-sqrdists, K)
    idx_base = jnp.arange(B)[:, None, None] * N
    flat_idx = (idx + idx_base).reshape(-1)
    nb = xyz.reshape(B * N, 3)[flat_idx, :].reshape(B, NG, K * 3)

    out = pl.pallas_call(
        _recenter_body,
        grid=(B,),
        in_specs=[
            pl.BlockSpec((1, NG, K * 3), lambda b: (b, 0, 0)),
            pl.BlockSpec((1, NG, 3), lambda b: (b, 0, 0)),
        ],
        out_specs=pl.BlockSpec((1, NG, K * 3), lambda b: (b, 0, 0)),
        out_shape=jax.ShapeDtypeStruct((B, NG, K * 3), jnp.float32),
    )(nb, center)
    return (out.reshape(B, NG, K, 3), center)


# speedup vs baseline: 1.8244x; 1.8244x over previous
import jax
import jax.numpy as jnp
from jax import lax
from jax.experimental import pallas as pl

NG = 512
K = 32


def _fps_body(x_ref, y_ref, z_ref, f0_ref, cidx_ref):
    B, N = x_ref.shape
    x = x_ref[...]
    y = y_ref[...]
    z = z_ref[...]
    lane = lax.broadcasted_iota(jnp.int32, (B, N), 1)
    col = lax.broadcasted_iota(jnp.int32, (B, NG), 1)

    def body(i, carry):
        distance, f, acc = carry
        acc = acc + jnp.where(col == i, jnp.broadcast_to(f, (B, NG)), 0)
        sel = lane == f
        cx = jnp.sum(jnp.where(sel, x, 0.0), axis=1, keepdims=True)
        cy = jnp.sum(jnp.where(sel, y, 0.0), axis=1, keepdims=True)
        cz = jnp.sum(jnp.where(sel, z, 0.0), axis=1, keepdims=True)
        dx = x - cx
        dy = y - cy
        dz = z - cz
        d = (dx * dx + dy * dy) + dz * dz
        distance = jnp.minimum(distance, d)
        m = jnp.max(distance, axis=1, keepdims=True)
        f_new = jnp.min(jnp.where(distance == m, lane, N), axis=1, keepdims=True)
        return (distance, f_new, acc)

    dist0 = jnp.full((B, N), 1e10, dtype=jnp.float32)
    acc0 = jnp.zeros((B, NG), dtype=jnp.int32)
    _, _, acc = lax.fori_loop(0, NG, body, (dist0, f0_ref[:, :1], acc0))
    cidx_ref[...] = acc


def _fps_pallas(xyz, npoint, key):
    B, N, _ = xyz.shape
    farthest0 = jax.random.randint(key, (B,), 0, N).astype(jnp.int32)
    xt = jnp.transpose(xyz, (2, 0, 1))
    f0 = jnp.broadcast_to(farthest0[:, None], (B, 128))
    cidx = pl.pallas_call(
        _fps_body,
        out_shape=jax.ShapeDtypeStruct((B, npoint), jnp.int32),
    )(xt[0], xt[1], xt[2], f0)
    return cidx


def _recenter_body(nb_ref, c_ref, out_ref):
    nb = nb_ref[0].reshape(NG, K, 3)
    out_ref[0] = (nb - c_ref[0][:, None, :]).reshape(NG, K * 3)


def kernel(xyz):
    B, N, C = xyz.shape
    centroids = _fps_pallas(xyz, NG, jax.random.fold_in(jax.random.key(0), 1))
    center = jnp.take_along_axis(xyz, centroids[:, :, None], axis=1)
    sqrdists = -2.0 * jnp.matmul(center, jnp.transpose(xyz, (0, 2, 1)))
    sqrdists = sqrdists + jnp.sum(center ** 2, axis=-1)[:, :, None]
    sqrdists = sqrdists + jnp.sum(xyz ** 2, axis=-1)[:, None, :]
    _, idx = lax.top_k(-sqrdists, K)
    idx_base = jnp.arange(B)[:, None, None] * N
    flat_idx = (idx + idx_base).reshape(-1)
    nb = xyz.reshape(B * N, 3)[flat_idx, :].reshape(B, NG, K * 3)

    out = pl.pallas_call(
        _recenter_body,
        grid=(B,),
        in_specs=[
            pl.BlockSpec((1, NG, K * 3), lambda b: (b, 0, 0)),
            pl.BlockSpec((1, NG, 3), lambda b: (b, 0, 0)),
        ],
        out_specs=pl.BlockSpec((1, NG, K * 3), lambda b: (b, 0, 0)),
        out_shape=jax.ShapeDtypeStruct((B, NG, K * 3), jnp.float32),
    )(nb, center)
    return (out.reshape(B, NG, K, 3), center)


# speedup vs baseline: 11.3447x; 6.2184x over previous
import functools

import jax
import jax.numpy as jnp
from jax import lax
from jax.experimental import pallas as pl
from jax.experimental.pallas import tpu as pltpu
from jax.experimental.pallas import tpu_sc as plsc

NG = 512
K = 32
L = 16
_NC = 2
_NS = 16


def _fps_body(x_ref, y_ref, z_ref, f0_ref, cidx_ref):
    B, N = x_ref.shape
    x = x_ref[...]
    y = y_ref[...]
    z = z_ref[...]
    lane = lax.broadcasted_iota(jnp.int32, (B, N), 1)
    col = lax.broadcasted_iota(jnp.int32, (B, NG), 1)

    def body(i, carry):
        distance, f, acc = carry
        acc = acc + jnp.where(col == i, jnp.broadcast_to(f, (B, NG)), 0)
        sel = lane == f
        cx = jnp.sum(jnp.where(sel, x, 0.0), axis=1, keepdims=True)
        cy = jnp.sum(jnp.where(sel, y, 0.0), axis=1, keepdims=True)
        cz = jnp.sum(jnp.where(sel, z, 0.0), axis=1, keepdims=True)
        dx = x - cx
        dy = y - cy
        dz = z - cz
        d = (dx * dx + dy * dy) + dz * dz
        distance = jnp.minimum(distance, d)
        m = jnp.max(distance, axis=1, keepdims=True)
        f_new = jnp.min(jnp.where(distance == m, lane, N), axis=1, keepdims=True)
        return (distance, f_new, acc)

    dist0 = jnp.full((B, N), 1e10, dtype=jnp.float32)
    acc0 = jnp.zeros((B, NG), dtype=jnp.int32)
    _, _, acc = lax.fori_loop(0, NG, body, (dist0, f0_ref[:, :1], acc0))
    cidx_ref[...] = acc


def _dist_body(c_ref, xt_ref, o_ref):
    c = c_ref[0]
    xt = xt_ref[0]
    mm = jnp.dot(c, xt, preferred_element_type=jnp.float32)
    dist = -2.0 * mm
    nc = jnp.sum(c ** 2, axis=-1)[:, None]
    npp = jnp.sum(xt ** 2, axis=0)[None, :]
    o_ref[0] = dist + nc + npp


def _lex_lt(k, i, pk, pi):
    return (k < pk) | ((k == pk) & (i < pi))


def _lex_sort16(k, v):
    vi, kk = plsc.sort_key_val(v, k)
    sk, sv = plsc.sort_key_val(kk, vi)
    return sk, sv


def _merge_topk(ak, ai, bk, bi, sk, si):
    rk = lax.rev(sk, (0,))
    ri = lax.rev(si, (0,))
    take_s = _lex_lt(rk, ri, bk, bi)
    bk2 = jnp.where(take_s, rk, bk)
    bi2 = jnp.where(take_s, ri, bi)
    lo = _lex_lt(bk2, bi2, ak, ai)
    nak = jnp.where(lo, bk2, ak)
    nai = jnp.where(lo, bi2, ai)
    nbk = jnp.where(lo, ak, bk2)
    nbi = jnp.where(lo, ai, bi2)
    nak, nai = _lex_sort16(nak, nai)
    nbk, nbi = _lex_sort16(nbk, nbi)
    return nak, nai, nbk, nbi


def _sc_body(d_hbm, xt_hbm, cidx_hbm, out_hbm,
             x_v, y_v, z_v, d_v, cd_v, ci_v, ci128_v, o_v):
    wid = lax.axis_index("s") * _NC + lax.axis_index("c")
    b = wid // 4
    q = wid % 4
    N = 8192

    pltpu.sync_copy(xt_hbm.at[0, b], x_v)
    pltpu.sync_copy(xt_hbm.at[1, b], y_v)
    pltpu.sync_copy(xt_hbm.at[2, b], z_v)
    pltpu.sync_copy(cidx_hbm.at[b, pl.ds(q * 128, 128)], ci128_v)

    inf16 = jnp.full((L,), jnp.inf, jnp.float32)
    bigi16 = jnp.full((L,), jnp.int32(2 ** 30), jnp.int32)
    iota16 = lax.iota(jnp.int32, L)

    def row_body(r, _):
        g = q * 128 + r
        pltpu.sync_copy(d_hbm.at[b, g], d_v)

        def pass1(j, carry):
            m1, m2 = carry
            d = d_v[pl.ds(j * L, L)]
            nm1 = jnp.minimum(m1, d)
            nm2 = jnp.minimum(m2, jnp.maximum(m1, d))
            return nm1, nm2

        m1, m2 = lax.fori_loop(0, N // L, pass1, (inf16, inf16))
        t = jnp.max(m2)

        def pass2(j, off):
            d = d_v[pl.ds(j * L, L)]
            mask = d <= t
            idx = iota16 + j * L
            offc = jnp.minimum(off, 512)
            plsc.store_compressed(cd_v.at[pl.ds(offc, L)], d, mask=mask)
            plsc.store_compressed(ci_v.at[pl.ds(offc, L)], idx, mask=mask)
            cnt = jnp.max(plsc.all_reduce_population_count(mask))
            return off + cnt

        off = lax.fori_loop(0, N // L, pass2, jnp.int32(0))
        offc = jnp.minimum(off, 512)
        cd_v[pl.ds(offc, L)] = inf16
        ci_v[pl.ds(offc, L)] = bigi16
        nv = (offc + (L - 1)) // L

        def merge_body(j, carry):
            ak, ai, bk, bi = carry
            sk, si = plsc.sort_key_val(cd_v[pl.ds(j * L, L)], ci_v[pl.ds(j * L, L)])
            return _merge_topk(ak, ai, bk, bi, sk, si)

        ak, ai, bk, bi = lax.fori_loop(
            0, nv, merge_body, (inf16, bigi16, inf16, bigi16))

        rv = jnp.full((L,), r, jnp.int32)
        civ = plsc.load_gather(ci128_v, [rv])
        cxv = plsc.load_gather(x_v, [civ])
        cyv = plsc.load_gather(y_v, [civ])
        czv = plsc.load_gather(z_v, [civ])
        base = r * 96
        o_v[pl.ds(base + 0, L)] = plsc.load_gather(x_v, [ai]) - cxv
        o_v[pl.ds(base + 16, L)] = plsc.load_gather(x_v, [bi]) - cxv
        o_v[pl.ds(base + 32, L)] = plsc.load_gather(y_v, [ai]) - cyv
        o_v[pl.ds(base + 48, L)] = plsc.load_gather(y_v, [bi]) - cyv
        o_v[pl.ds(base + 64, L)] = plsc.load_gather(z_v, [ai]) - czv
        o_v[pl.ds(base + 80, L)] = plsc.load_gather(z_v, [bi]) - czv
        return 0

    lax.fori_loop(0, 128, row_body, 0)
    pltpu.sync_copy(o_v, out_hbm.at[pl.ds(wid * 12288, 12288)])


def kernel(xyz):
    B, N, C = xyz.shape
    key = jax.random.fold_in(jax.random.key(0), 1)
    farthest0 = jax.random.randint(key, (B,), 0, N).astype(jnp.int32)
    xt = jnp.transpose(xyz, (2, 0, 1))
    f0 = jnp.broadcast_to(farthest0[:, None], (B, 128))

    cidx = pl.pallas_call(
        _fps_body,
        out_shape=jax.ShapeDtypeStruct((B, NG), jnp.int32),
    )(xt[0], xt[1], xt[2], f0)

    center = jnp.take_along_axis(xyz, cidx[:, :, None], axis=1)

    xtb = jnp.transpose(xyz, (0, 2, 1))
    d = pl.pallas_call(
        _dist_body,
        grid=(B, NG // 64),
        in_specs=[
            pl.BlockSpec((1, 64, 3), lambda b, g: (b, g, 0)),
            pl.BlockSpec((1, 3, N), lambda b, g: (b, 0, 0)),
        ],
        out_specs=pl.BlockSpec((1, 64, N), lambda b, g: (b, g, 0)),
        out_shape=jax.ShapeDtypeStruct((B, NG, N), jnp.float32),
    )(center, xtb)

    mesh = plsc.VectorSubcoreMesh(
        core_axis_name="c", subcore_axis_name="s",
        num_cores=_NC, num_subcores=_NS)
    sc = functools.partial(
        pl.kernel,
        out_type=jax.ShapeDtypeStruct((B * NG * 96,), jnp.float32),
        mesh=mesh,
        compiler_params=pltpu.CompilerParams(needs_layout_passes=False),
        scratch_types=[
            pltpu.VMEM((N,), jnp.float32),
            pltpu.VMEM((N,), jnp.float32),
            pltpu.VMEM((N,), jnp.float32),
            pltpu.VMEM((N,), jnp.float32),
            pltpu.VMEM((544,), jnp.float32),
            pltpu.VMEM((544,), jnp.int32),
            pltpu.VMEM((128,), jnp.int32),
            pltpu.VMEM((12288,), jnp.float32),
        ],
    )(_sc_body)
    flat = sc(d, xt, cidx)
    neighborhood = flat.reshape(B, NG, 3, K).transpose(0, 1, 3, 2)
    return (neighborhood, center)


# speedup vs baseline: 13.5036x; 1.1903x over previous
import functools

import jax
import jax.numpy as jnp
from jax import lax
from jax.experimental import pallas as pl
from jax.experimental.pallas import tpu as pltpu
from jax.experimental.pallas import tpu_sc as plsc

NG = 512
K = 32
L = 16
_NC = 2
_NS = 16


def _fps_body(x_ref, y_ref, z_ref, f0_ref, cidx_ref):
    B, N = x_ref.shape
    x = x_ref[...]
    y = y_ref[...]
    z = z_ref[...]
    lane = lax.broadcasted_iota(jnp.int32, (B, N), 1)
    col = lax.broadcasted_iota(jnp.int32, (B, NG), 1)

    def body(i, carry):
        distance, f, acc = carry
        acc = acc + jnp.where(col == i, jnp.broadcast_to(f, (B, NG)), 0)
        sel = lane == f
        cx = jnp.sum(jnp.where(sel, x, 0.0), axis=1, keepdims=True)
        cy = jnp.sum(jnp.where(sel, y, 0.0), axis=1, keepdims=True)
        cz = jnp.sum(jnp.where(sel, z, 0.0), axis=1, keepdims=True)
        dx = x - cx
        dy = y - cy
        dz = z - cz
        d = (dx * dx + dy * dy) + dz * dz
        distance = jnp.minimum(distance, d)
        m = jnp.max(distance, axis=1, keepdims=True)
        f_new = jnp.min(jnp.where(distance == m, lane, N), axis=1, keepdims=True)
        return (distance, f_new, acc)

    dist0 = jnp.full((B, N), 1e10, dtype=jnp.float32)
    acc0 = jnp.zeros((B, NG), dtype=jnp.int32)
    _, _, acc = lax.fori_loop(0, NG, body, (dist0, f0_ref[:, :1], acc0))
    cidx_ref[...] = acc


def _dist_body(c_ref, xt_ref, o_ref):
    c = c_ref[0]
    xt = xt_ref[0]
    mm = jnp.dot(c, xt, preferred_element_type=jnp.float32)
    dist = -2.0 * mm
    cx = c[:, 0:1]
    cy = c[:, 1:2]
    cz = c[:, 2:3]
    nc = (cx * cx + cy * cy) + cz * cz
    x = xt[0:1, :]
    y = xt[1:2, :]
    z = xt[2:3, :]
    npp = (x * x + y * y) + z * z
    o_ref[0] = dist + nc + npp


def _lex_lt(k, i, pk, pi):
    return (k < pk) | ((k == pk) & (i < pi))


def _lex_sort16(k, v):
    vi, kk = plsc.sort_key_val(v, k)
    sk, sv = plsc.sort_key_val(kk, vi)
    return sk, sv


def _merge_topk(ak, ai, bk, bi, sk, si):
    rk = lax.rev(sk, (0,))
    ri = lax.rev(si, (0,))
    take_s = _lex_lt(rk, ri, bk, bi)
    bk2 = jnp.where(take_s, rk, bk)
    bi2 = jnp.where(take_s, ri, bi)
    lo = _lex_lt(bk2, bi2, ak, ai)
    nak = jnp.where(lo, bk2, ak)
    nai = jnp.where(lo, bi2, ai)
    nbk = jnp.where(lo, ak, bk2)
    nbi = jnp.where(lo, ai, bi2)
    nak, nai = _lex_sort16(nak, nai)
    nbk, nbi = _lex_sort16(nbk, nbi)
    return nak, nai, nbk, nbi


def _sc_body(d_hbm, xt_hbm, cidx_hbm, out_hbm,
             x_v, y_v, z_v, d_v, d_v2, cd_v, ci_v, ci128_v, o_v,
             sem0, sem1):
    wid = lax.axis_index("s") * _NC + lax.axis_index("c")
    b = wid // 4
    q = wid % 4
    N = 8192
    g0 = q * 128

    pltpu.sync_copy(xt_hbm.at[0, b], x_v)
    pltpu.sync_copy(xt_hbm.at[1, b], y_v)
    pltpu.sync_copy(xt_hbm.at[2, b], z_v)
    pltpu.sync_copy(cidx_hbm.at[b, pl.ds(g0, 128)], ci128_v)

    inf16 = jnp.full((L,), jnp.inf, jnp.float32)
    bigi16 = jnp.full((L,), jnp.int32(2 ** 30), jnp.int32)
    iota16 = lax.iota(jnp.int32, L)

    def process_row(r, buf):
        def pass1(j, carry):
            m1a, m2a, m1b, m2b = carry
            base = j * 4
            d0 = buf[pl.ds((base + 0) * L, L)]
            d1 = buf[pl.ds((base + 1) * L, L)]
            d2 = buf[pl.ds((base + 2) * L, L)]
            d3 = buf[pl.ds((base + 3) * L, L)]
            n1a = jnp.minimum(m1a, d0)
            m2a = jnp.minimum(m2a, jnp.maximum(m1a, d0))
            m1a = n1a
            n1a = jnp.minimum(m1a, d1)
            m2a = jnp.minimum(m2a, jnp.maximum(m1a, d1))
            m1a = n1a
            n1b = jnp.minimum(m1b, d2)
            m2b = jnp.minimum(m2b, jnp.maximum(m1b, d2))
            m1b = n1b
            n1b = jnp.minimum(m1b, d3)
            m2b = jnp.minimum(m2b, jnp.maximum(m1b, d3))
            m1b = n1b
            return m1a, m2a, m1b, m2b

        m1a, m2a, m1b, m2b = lax.fori_loop(
            0, N // (4 * L), pass1, (inf16, inf16, inf16, inf16))
        m2 = jnp.minimum(jnp.minimum(m2a, m2b), jnp.maximum(m1a, m1b))
        t = jnp.max(m2)

        def pass2(j, off):
            base = j * 4
            for u in range(4):
                d = buf[pl.ds((base + u) * L, L)]
                mask = d <= t
                idx = iota16 + (base + u) * L
                offc = jnp.minimum(off, 512)
                plsc.store_compressed(cd_v.at[pl.ds(offc, L)], d, mask=mask)
                plsc.store_compressed(ci_v.at[pl.ds(offc, L)], idx, mask=mask)
                off = off + jnp.max(plsc.all_reduce_population_count(mask))
            return off

        off = lax.fori_loop(0, N // (4 * L), pass2, jnp.int32(0))
        offc = jnp.minimum(off, 512)
        cd_v[pl.ds(offc, L)] = inf16
        ci_v[pl.ds(offc, L)] = bigi16
        nv = (offc + (L - 1)) // L

        def merge_body(j, carry):
            ak, ai, bk, bi = carry
            sk, si = plsc.sort_key_val(cd_v[pl.ds(j * L, L)], ci_v[pl.ds(j * L, L)])
            return _merge_topk(ak, ai, bk, bi, sk, si)

        ak, ai, bk, bi = lax.fori_loop(
            0, nv, merge_body, (inf16, bigi16, inf16, bigi16))

        rv = jnp.full((L,), r, jnp.int32)
        civ = plsc.load_gather(ci128_v, [rv])
        cxv = plsc.load_gather(x_v, [civ])
        cyv = plsc.load_gather(y_v, [civ])
        czv = plsc.load_gather(z_v, [civ])
        base = r * 96
        o_v[pl.ds(base + 0, L)] = plsc.load_gather(x_v, [ai]) - cxv
        o_v[pl.ds(base + 16, L)] = plsc.load_gather(x_v, [bi]) - cxv
        o_v[pl.ds(base + 32, L)] = plsc.load_gather(y_v, [ai]) - cyv
        o_v[pl.ds(base + 48, L)] = plsc.load_gather(y_v, [bi]) - cyv
        o_v[pl.ds(base + 64, L)] = plsc.load_gather(z_v, [ai]) - czv
        o_v[pl.ds(base + 80, L)] = plsc.load_gather(z_v, [bi]) - czv

    bufs = (d_v, d_v2)
    sems = (sem0, sem1)
    pltpu.async_copy(d_hbm.at[b, g0], d_v, sem0)

    def outer(g, c):
        for u in range(2):
            r = g * 2 + u
            buf, sem = bufs[u], sems[u]
            nbuf, nsem = bufs[1 - u], sems[1 - u]
            nr = (r + 1) % 128
            pltpu.async_copy(d_hbm.at[b, g0 + nr], nbuf, nsem)
            pltpu.make_async_copy(d_hbm.at[b, g0 + r], buf, sem).wait()
            process_row(r, buf)
        return c

    lax.fori_loop(0, 64, outer, 0)
    pltpu.make_async_copy(d_hbm.at[b, g0], d_v, sem0).wait()
    pltpu.sync_copy(o_v, out_hbm.at[pl.ds(wid * 12288, 12288)])


def kernel(xyz):
    B, N, C = xyz.shape
    key = jax.random.fold_in(jax.random.key(0), 1)
    farthest0 = jax.random.randint(key, (B,), 0, N).astype(jnp.int32)
    xt = jnp.transpose(xyz, (2, 0, 1))
    f0 = jnp.broadcast_to(farthest0[:, None], (B, 128))

    cidx = pl.pallas_call(
        _fps_body,
        out_shape=jax.ShapeDtypeStruct((B, NG), jnp.int32),
    )(xt[0], xt[1], xt[2], f0)

    center = jnp.take_along_axis(xyz, cidx[:, :, None], axis=1)

    xtb = jnp.transpose(xyz, (0, 2, 1))
    d = pl.pallas_call(
        _dist_body,
        grid=(B, NG // 64),
        in_specs=[
            pl.BlockSpec((1, 64, 3), lambda b, g: (b, g, 0)),
            pl.BlockSpec((1, 3, N), lambda b, g: (b, 0, 0)),
        ],
        out_specs=pl.BlockSpec((1, 64, N), lambda b, g: (b, g, 0)),
        out_shape=jax.ShapeDtypeStruct((B, NG, N), jnp.float32),
    )(center, xtb)

    mesh = plsc.VectorSubcoreMesh(
        core_axis_name="c", subcore_axis_name="s",
        num_cores=_NC, num_subcores=_NS)
    sc = functools.partial(
        pl.kernel,
        out_type=jax.ShapeDtypeStruct((B * NG * 96,), jnp.float32),
        mesh=mesh,
        compiler_params=pltpu.CompilerParams(needs_layout_passes=False),
        scratch_types=[
            pltpu.VMEM((N,), jnp.float32),
            pltpu.VMEM((N,), jnp.float32),
            pltpu.VMEM((N,), jnp.float32),
            pltpu.VMEM((N,), jnp.float32),
            pltpu.VMEM((N,), jnp.float32),
            pltpu.VMEM((544,), jnp.float32),
            pltpu.VMEM((544,), jnp.int32),
            pltpu.VMEM((128,), jnp.int32),
            pltpu.VMEM((12288,), jnp.float32),
            pltpu.SemaphoreType.DMA,
            pltpu.SemaphoreType.DMA,
        ],
    )(_sc_body)
    flat = sc(d, xt, cidx)
    neighborhood = flat.reshape(B, NG, 3, K).transpose(0, 1, 3, 2)
    return (neighborhood, center)


# speedup vs baseline: 15.3008x; 1.1331x over previous
import functools

import jax
import jax.numpy as jnp
from jax import lax
from jax.experimental import pallas as pl
from jax.experimental.pallas import tpu as pltpu
from jax.experimental.pallas import tpu_sc as plsc

NG = 512
K = 32
L = 16
_NC = 2
_NS = 16


def _fps_body(x_ref, y_ref, z_ref, f0_ref, cidx_ref):
    B, N = x_ref.shape
    x = x_ref[...]
    y = y_ref[...]
    z = z_ref[...]
    lane = lax.broadcasted_iota(jnp.int32, (B, N), 1)
    col = lax.broadcasted_iota(jnp.int32, (B, NG), 1)

    def body(i, carry):
        distance, f, acc = carry
        acc = acc + jnp.where(col == i, jnp.broadcast_to(f, (B, NG)), 0)
        sel = lane == f
        cx = jnp.sum(jnp.where(sel, x, 0.0), axis=1, keepdims=True)
        cy = jnp.sum(jnp.where(sel, y, 0.0), axis=1, keepdims=True)
        cz = jnp.sum(jnp.where(sel, z, 0.0), axis=1, keepdims=True)
        dx = x - cx
        dy = y - cy
        dz = z - cz
        d = (dx * dx + dy * dy) + dz * dz
        distance = jnp.minimum(distance, d)
        m = jnp.max(distance, axis=1, keepdims=True)
        f_new = jnp.min(jnp.where(distance == m, lane, N), axis=1, keepdims=True)
        return (distance, f_new, acc)

    dist0 = jnp.full((B, N), 1e10, dtype=jnp.float32)
    acc0 = jnp.zeros((B, NG), dtype=jnp.int32)
    _, _, acc = lax.fori_loop(0, NG, body, (dist0, f0_ref[:, :1], acc0))
    cidx_ref[...] = acc


def _dist_body(c_ref, xt_ref, o_ref):
    c = c_ref[0]
    xt = xt_ref[0]
    mm = jnp.dot(c, xt, preferred_element_type=jnp.float32)
    dist = -2.0 * mm
    cx = c[:, 0:1]
    cy = c[:, 1:2]
    cz = c[:, 2:3]
    nc = (cx * cx + cy * cy) + cz * cz
    x = xt[0:1, :]
    y = xt[1:2, :]
    z = xt[2:3, :]
    npp = (x * x + y * y) + z * z
    o_ref[0] = dist + nc + npp


def _lex_lt(k, i, pk, pi):
    return (k < pk) | ((k == pk) & (i < pi))


def _lex_sort16(k, v):
    vi, kk = plsc.sort_key_val(v, k)
    sk, sv = plsc.sort_key_val(kk, vi)
    return sk, sv


def _merge_topk(ak, ai, bk, bi, sk, si):
    rk = lax.rev(sk, (0,))
    ri = lax.rev(si, (0,))
    take_s = _lex_lt(rk, ri, bk, bi)
    bk2 = jnp.where(take_s, rk, bk)
    bi2 = jnp.where(take_s, ri, bi)
    lo = _lex_lt(bk2, bi2, ak, ai)
    nak = jnp.where(lo, bk2, ak)
    nai = jnp.where(lo, bi2, ai)
    nbk = jnp.where(lo, ak, bk2)
    nbi = jnp.where(lo, ai, bi2)
    nak, nai = _lex_sort16(nak, nai)
    nbk, nbi = _lex_sort16(nbk, nbi)
    return nak, nai, nbk, nbi


def _sc_body(d_hbm, xt_hbm, cidx_hbm, out_hbm,
             x_v, y_v, z_v, d_v, d_v2, cd_v, ci_v, ci128_v, o_v,
             sem0, sem1):
    wid = lax.axis_index("s") * _NC + lax.axis_index("c")
    b = wid // 4
    q = wid % 4
    N = 8192
    g0 = q * 128

    pltpu.sync_copy(xt_hbm.at[0, b], x_v)
    pltpu.sync_copy(xt_hbm.at[1, b], y_v)
    pltpu.sync_copy(xt_hbm.at[2, b], z_v)
    pltpu.sync_copy(cidx_hbm.at[b, pl.ds(g0, 128)], ci128_v)

    inf16 = jnp.full((L,), jnp.inf, jnp.float32)
    bigi16 = jnp.full((L,), jnp.int32(2 ** 30), jnp.int32)
    iota16 = lax.iota(jnp.int32, L)

    def process_row(r, buf):
        def pass1(j, carry):
            m1a, m2a, m1b, m2b = carry
            base = j * 4
            d0 = buf[pl.ds((base + 0) * L, L)]
            d1 = buf[pl.ds((base + 1) * L, L)]
            d2 = buf[pl.ds((base + 2) * L, L)]
            d3 = buf[pl.ds((base + 3) * L, L)]
            n1a = jnp.minimum(m1a, d0)
            m2a = jnp.minimum(m2a, jnp.maximum(m1a, d0))
            m1a = n1a
            n1a = jnp.minimum(m1a, d1)
            m2a = jnp.minimum(m2a, jnp.maximum(m1a, d1))
            m1a = n1a
            n1b = jnp.minimum(m1b, d2)
            m2b = jnp.minimum(m2b, jnp.maximum(m1b, d2))
            m1b = n1b
            n1b = jnp.minimum(m1b, d3)
            m2b = jnp.minimum(m2b, jnp.maximum(m1b, d3))
            m1b = n1b
            return m1a, m2a, m1b, m2b

        m1a, m2a, m1b, m2b = lax.fori_loop(
            0, N // (4 * L), pass1, (inf16, inf16, inf16, inf16))
        m2 = jnp.minimum(jnp.minimum(m2a, m2b), jnp.maximum(m1a, m1b))
        t = jnp.max(m2)

        def pass2(j, off):
            base = j * 4
            for u in range(4):
                d = buf[pl.ds((base + u) * L, L)]
                mask = d <= t
                idx = iota16 + (base + u) * L
                offc = jnp.minimum(off, 512)
                plsc.store_compressed(cd_v.at[pl.ds(offc, L)], d, mask=mask)
                plsc.store_compressed(ci_v.at[pl.ds(offc, L)], idx, mask=mask)
                off = off + plsc.all_reduce_population_count(mask)[0]
            return off

        off = lax.fori_loop(0, N // (4 * L), pass2, jnp.int32(0))
        offc = jnp.minimum(off, 512)
        cd_v[pl.ds(offc, L)] = inf16
        ci_v[pl.ds(offc, L)] = bigi16
        nv = (offc + (L - 1)) // L

        def merge_body(j, carry):
            ak, ai, bk, bi = carry
            sk, si = plsc.sort_key_val(cd_v[pl.ds(j * L, L)], ci_v[pl.ds(j * L, L)])
            return _merge_topk(ak, ai, bk, bi, sk, si)

        ak, ai, bk, bi = lax.fori_loop(
            0, nv, merge_body, (inf16, bigi16, inf16, bigi16))

        rv = jnp.full((L,), r, jnp.int32)
        civ = plsc.load_gather(ci128_v, [rv])
        cxv = plsc.load_gather(x_v, [civ])
        cyv = plsc.load_gather(y_v, [civ])
        czv = plsc.load_gather(z_v, [civ])
        base = r * 96
        o_v[pl.ds(base + 0, L)] = plsc.load_gather(x_v, [ai]) - cxv
        o_v[pl.ds(base + 16, L)] = plsc.load_gather(x_v, [bi]) - cxv
        o_v[pl.ds(base + 32, L)] = plsc.load_gather(y_v, [ai]) - cyv
        o_v[pl.ds(base + 48, L)] = plsc.load_gather(y_v, [bi]) - cyv
        o_v[pl.ds(base + 64, L)] = plsc.load_gather(z_v, [ai]) - czv
        o_v[pl.ds(base + 80, L)] = plsc.load_gather(z_v, [bi]) - czv

    bufs = (d_v, d_v2)
    sems = (sem0, sem1)
    pltpu.async_copy(d_hbm.at[b, g0], d_v, sem0)

    def outer(g, c):
        for u in range(2):
            r = g * 2 + u
            buf, sem = bufs[u], sems[u]
            nbuf, nsem = bufs[1 - u], sems[1 - u]
            nr = (r + 1) % 128
            pltpu.async_copy(d_hbm.at[b, g0 + nr], nbuf, nsem)
            pltpu.make_async_copy(d_hbm.at[b, g0 + r], buf, sem).wait()
            process_row(r, buf)
        return c

    lax.fori_loop(0, 64, outer, 0)
    pltpu.make_async_copy(d_hbm.at[b, g0], d_v, sem0).wait()
    pltpu.sync_copy(o_v, out_hbm.at[pl.ds(wid * 12288, 12288)])


def kernel(xyz):
    B, N, C = xyz.shape
    key = jax.random.fold_in(jax.random.key(0), 1)
    farthest0 = jax.random.randint(key, (B,), 0, N).astype(jnp.int32)
    xt = jnp.transpose(xyz, (2, 0, 1))
    f0 = jnp.broadcast_to(farthest0[:, None], (B, 128))

    cidx = pl.pallas_call(
        _fps_body,
        out_shape=jax.ShapeDtypeStruct((B, NG), jnp.int32),
    )(xt[0], xt[1], xt[2], f0)

    center = jnp.take_along_axis(xyz, cidx[:, :, None], axis=1)

    xtb = jnp.transpose(xyz, (0, 2, 1))
    d = pl.pallas_call(
        _dist_body,
        grid=(B, NG // 64),
        in_specs=[
            pl.BlockSpec((1, 64, 3), lambda b, g: (b, g, 0)),
            pl.BlockSpec((1, 3, N), lambda b, g: (b, 0, 0)),
        ],
        out_specs=pl.BlockSpec((1, 64, N), lambda b, g: (b, g, 0)),
        out_shape=jax.ShapeDtypeStruct((B, NG, N), jnp.float32),
    )(center, xtb)

    mesh = plsc.VectorSubcoreMesh(
        core_axis_name="c", subcore_axis_name="s",
        num_cores=_NC, num_subcores=_NS)
    sc = functools.partial(
        pl.kernel,
        out_type=jax.ShapeDtypeStruct((B * NG * 96,), jnp.float32),
        mesh=mesh,
        compiler_params=pltpu.CompilerParams(needs_layout_passes=False),
        scratch_types=[
            pltpu.VMEM((N,), jnp.float32),
            pltpu.VMEM((N,), jnp.float32),
            pltpu.VMEM((N,), jnp.float32),
            pltpu.VMEM((N,), jnp.float32),
            pltpu.VMEM((N,), jnp.float32),
            pltpu.VMEM((544,), jnp.float32),
            pltpu.VMEM((544,), jnp.int32),
            pltpu.VMEM((128,), jnp.int32),
            pltpu.VMEM((12288,), jnp.float32),
            pltpu.SemaphoreType.DMA,
            pltpu.SemaphoreType.DMA,
        ],
    )(_sc_body)
    flat = sc(d, xt, cidx)
    neighborhood = flat.reshape(B, NG, 3, K).transpose(0, 1, 3, 2)
    return (neighborhood, center)


# speedup vs baseline: 15.6510x; 1.0229x over previous
import functools

import jax
import jax.numpy as jnp
from jax import lax
from jax.experimental import pallas as pl
from jax.experimental.pallas import tpu as pltpu
from jax.experimental.pallas import tpu_sc as plsc

NG = 512
K = 32
L = 16
_NC = 2
_NS = 16


def _fps_body(x_ref, y_ref, z_ref, f0_ref, cidx_ref):
    B, N = x_ref.shape
    x = x_ref[...]
    y = y_ref[...]
    z = z_ref[...]
    lane = lax.broadcasted_iota(jnp.int32, (B, N), 1)
    col = lax.broadcasted_iota(jnp.int32, (B, NG), 1)

    def body(i, carry):
        distance, f, acc = carry
        acc = acc + jnp.where(col == i, jnp.broadcast_to(f, (B, NG)), 0)
        sel = lane == f
        cx = jnp.sum(jnp.where(sel, x, 0.0), axis=1, keepdims=True)
        cy = jnp.sum(jnp.where(sel, y, 0.0), axis=1, keepdims=True)
        cz = jnp.sum(jnp.where(sel, z, 0.0), axis=1, keepdims=True)
        dx = x - cx
        dy = y - cy
        dz = z - cz
        d = (dx * dx + dy * dy) + dz * dz
        distance = jnp.minimum(distance, d)
        m = jnp.max(distance, axis=1, keepdims=True)
        f_new = jnp.min(jnp.where(distance == m, lane, N), axis=1, keepdims=True)
        return (distance, f_new, acc)

    dist0 = jnp.full((B, N), 1e10, dtype=jnp.float32)
    acc0 = jnp.zeros((B, NG), dtype=jnp.int32)
    _, _, acc = lax.fori_loop(0, NG, body, (dist0, f0_ref[:, :1], acc0))
    cidx_ref[...] = acc


def _dist_body(c_ref, xt_ref, o_ref):
    c = c_ref[0]
    xt = xt_ref[0]
    mm = jnp.dot(c, xt, preferred_element_type=jnp.float32)
    dist = -2.0 * mm
    cx = c[:, 0:1]
    cy = c[:, 1:2]
    cz = c[:, 2:3]
    nc = (cx * cx + cy * cy) + cz * cz
    x = xt[0:1, :]
    y = xt[1:2, :]
    z = xt[2:3, :]
    npp = (x * x + y * y) + z * z
    o_ref[0] = dist + nc + npp


def _lex_lt(k, i, pk, pi):
    return (k < pk) | ((k == pk) & (i < pi))


def _lex_sort16(k, v):
    vi, kk = plsc.sort_key_val(v, k)
    sk, sv = plsc.sort_key_val(kk, vi)
    return sk, sv


def _merge_topk(ak, ai, bk, bi, sk, si):
    rk = lax.rev(sk, (0,))
    ri = lax.rev(si, (0,))
    take_s = _lex_lt(rk, ri, bk, bi)
    bk2 = jnp.where(take_s, rk, bk)
    bi2 = jnp.where(take_s, ri, bi)
    lo = _lex_lt(bk2, bi2, ak, ai)
    nak = jnp.where(lo, bk2, ak)
    nai = jnp.where(lo, bi2, ai)
    nbk = jnp.where(lo, ak, bk2)
    nbi = jnp.where(lo, ai, bi2)
    nak, nai = _lex_sort16(nak, nai)
    nbk, nbi = _lex_sort16(nbk, nbi)
    return nak, nai, nbk, nbi


def _sc_body(d_hbm, xt_hbm, cidx_hbm, out_hbm,
             x_v, y_v, z_v, d_v, d_v2, cd_v, ci_v, ci128_v, o_v,
             sem0, sem1):
    wid = lax.axis_index("s") * _NC + lax.axis_index("c")
    b = wid // 4
    q = wid % 4
    N = 8192
    g0 = q * 128

    pltpu.sync_copy(xt_hbm.at[0, b], x_v)
    pltpu.sync_copy(xt_hbm.at[1, b], y_v)
    pltpu.sync_copy(xt_hbm.at[2, b], z_v)
    pltpu.sync_copy(cidx_hbm.at[b, pl.ds(g0, 128)], ci128_v)

    inf16 = jnp.full((L,), jnp.inf, jnp.float32)
    bigi16 = jnp.full((L,), jnp.int32(2 ** 30), jnp.int32)
    iota16 = lax.iota(jnp.int32, L)

    def process_row(r, buf):
        def pass1(j, carry):
            m1a, m2a, m1b, m2b = carry
            base = j * 8
            for u in range(0, 8, 2):
                da = buf[pl.ds((base + u) * L, L)]
                db = buf[pl.ds((base + u + 1) * L, L)]
                n1a = jnp.minimum(m1a, da)
                m2a = jnp.minimum(m2a, jnp.maximum(m1a, da))
                m1a = n1a
                n1b = jnp.minimum(m1b, db)
                m2b = jnp.minimum(m2b, jnp.maximum(m1b, db))
                m1b = n1b
            return m1a, m2a, m1b, m2b

        m1a, m2a, m1b, m2b = lax.fori_loop(
            0, N // (8 * L), pass1, (inf16, inf16, inf16, inf16))
        m2 = jnp.minimum(jnp.minimum(m2a, m2b), jnp.maximum(m1a, m1b))
        t = jnp.max(m2)

        def pass2(j, off):
            base = j * 8
            for u in range(8):
                d = buf[pl.ds((base + u) * L, L)]
                mask = d <= t
                idx = iota16 + (base + u) * L
                offc = jnp.minimum(off, 512)
                plsc.store_compressed(cd_v.at[pl.ds(offc, L)], d, mask=mask)
                plsc.store_compressed(ci_v.at[pl.ds(offc, L)], idx, mask=mask)
                off = off + plsc.all_reduce_population_count(mask)[0]
            return off

        off = lax.fori_loop(0, N // (8 * L), pass2, jnp.int32(0))
        offc = jnp.minimum(off, 512)
        cd_v[pl.ds(offc, L)] = inf16
        ci_v[pl.ds(offc, L)] = bigi16
        nv = (offc + (L - 1)) // L

        def merge_body(j, carry):
            ak, ai, bk, bi = carry
            sk, si = plsc.sort_key_val(cd_v[pl.ds(j * L, L)], ci_v[pl.ds(j * L, L)])
            return _merge_topk(ak, ai, bk, bi, sk, si)

        ak, ai, bk, bi = lax.fori_loop(
            0, nv, merge_body, (inf16, bigi16, inf16, bigi16))

        rv = jnp.full((L,), r, jnp.int32)
        civ = plsc.load_gather(ci128_v, [rv])
        cxv = plsc.load_gather(x_v, [civ])
        cyv = plsc.load_gather(y_v, [civ])
        czv = plsc.load_gather(z_v, [civ])
        base = r * 96
        o_v[pl.ds(base + 0, L)] = plsc.load_gather(x_v, [ai]) - cxv
        o_v[pl.ds(base + 16, L)] = plsc.load_gather(x_v, [bi]) - cxv
        o_v[pl.ds(base + 32, L)] = plsc.load_gather(y_v, [ai]) - cyv
        o_v[pl.ds(base + 48, L)] = plsc.load_gather(y_v, [bi]) - cyv
        o_v[pl.ds(base + 64, L)] = plsc.load_gather(z_v, [ai]) - czv
        o_v[pl.ds(base + 80, L)] = plsc.load_gather(z_v, [bi]) - czv

    bufs = (d_v, d_v2)
    sems = (sem0, sem1)
    pltpu.async_copy(d_hbm.at[b, g0], d_v, sem0)

    def outer(g, c):
        for u in range(2):
            r = g * 2 + u
            buf, sem = bufs[u], sems[u]
            nbuf, nsem = bufs[1 - u], sems[1 - u]
            nr = (r + 1) % 128
            pltpu.async_copy(d_hbm.at[b, g0 + nr], nbuf, nsem)
            pltpu.make_async_copy(d_hbm.at[b, g0 + r], buf, sem).wait()
            process_row(r, buf)
        return c

    lax.fori_loop(0, 64, outer, 0)
    pltpu.make_async_copy(d_hbm.at[b, g0], d_v, sem0).wait()
    pltpu.sync_copy(o_v, out_hbm.at[pl.ds(wid * 12288, 12288)])


def kernel(xyz):
    B, N, C = xyz.shape
    key = jax.random.fold_in(jax.random.key(0), 1)
    farthest0 = jax.random.randint(key, (B,), 0, N).astype(jnp.int32)
    xt = jnp.transpose(xyz, (2, 0, 1))
    f0 = jnp.broadcast_to(farthest0[:, None], (B, 128))

    cidx = pl.pallas_call(
        _fps_body,
        out_shape=jax.ShapeDtypeStruct((B, NG), jnp.int32),
    )(xt[0], xt[1], xt[2], f0)

    center = jnp.take_along_axis(xyz, cidx[:, :, None], axis=1)

    xtb = jnp.transpose(xyz, (0, 2, 1))
    d = pl.pallas_call(
        _dist_body,
        grid=(B, NG // 64),
        in_specs=[
            pl.BlockSpec((1, 64, 3), lambda b, g: (b, g, 0)),
            pl.BlockSpec((1, 3, N), lambda b, g: (b, 0, 0)),
        ],
        out_specs=pl.BlockSpec((1, 64, N), lambda b, g: (b, g, 0)),
        out_shape=jax.ShapeDtypeStruct((B, NG, N), jnp.float32),
    )(center, xtb)

    mesh = plsc.VectorSubcoreMesh(
        core_axis_name="c", subcore_axis_name="s",
        num_cores=_NC, num_subcores=_NS)
    sc = functools.partial(
        pl.kernel,
        out_type=jax.ShapeDtypeStruct((B * NG * 96,), jnp.float32),
        mesh=mesh,
        compiler_params=pltpu.CompilerParams(needs_layout_passes=False),
        scratch_types=[
            pltpu.VMEM((N,), jnp.float32),
            pltpu.VMEM((N,), jnp.float32),
            pltpu.VMEM((N,), jnp.float32),
            pltpu.VMEM((N,), jnp.float32),
            pltpu.VMEM((N,), jnp.float32),
            pltpu.VMEM((544,), jnp.float32),
            pltpu.VMEM((544,), jnp.int32),
            pltpu.VMEM((128,), jnp.int32),
            pltpu.VMEM((12288,), jnp.float32),
            pltpu.SemaphoreType.DMA,
            pltpu.SemaphoreType.DMA,
        ],
    )(_sc_body)
    flat = sc(d, xt, cidx)
    neighborhood = flat.reshape(B, NG, 3, K).transpose(0, 1, 3, 2)
    return (neighborhood, center)
